# Initial kernel scaffold; baseline (speedup 1.0000x reference)
import jax
import jax.numpy as jnp
from jax.experimental import pallas as pl


def kernel(x, table):
    raise NotImplementedError("write your pallas kernel here")


# speedup vs baseline: 1.9855x; 1.9855x over previous
import functools

import jax
import jax.numpy as jnp
from jax import lax
from jax.experimental import pallas as pl
from jax.experimental.pallas import tpu as pltpu
from jax.experimental.pallas import tpu_sc as plsc


def _make_sc_gather(B, D, NW, NC, C):
    b_per_w = B // NW
    nchunks = b_per_w // C
    mesh = plsc.VectorSubcoreMesh(core_axis_name="c", subcore_axis_name="s")

    @functools.partial(
        pl.kernel,
        mesh=mesh,
        out_type=jax.ShapeDtypeStruct((B, D), jnp.float32),
        scratch_types=[
            pltpu.VMEM((b_per_w,), jnp.int32),
            pltpu.VMEM((C, D), jnp.float32),
            pltpu.SemaphoreType.DMA,
        ],
    )
    def k(idx_hbm, table_hbm, out_hbm, idx_v, rows_v, gsem):
        wid = lax.axis_index("s") * NC + lax.axis_index("c")
        base = wid * b_per_w
        pltpu.sync_copy(idx_hbm.at[pl.ds(base, b_per_w)], idx_v)

        def body(c, carry):
            off = c * C
            pltpu.async_copy(
                table_hbm.at[idx_v.at[pl.ds(off, C)]], rows_v, gsem
            ).wait()
            pltpu.sync_copy(rows_v, out_hbm.at[pl.ds(base + off, C)])
            return carry

        lax.fori_loop(0, nchunks, body, 0)

    return k


def kernel(x, table):
    batch, seq = x.shape
    max_len, d = table.shape
    B = batch * seq
    info = plsc.get_sparse_core_info()
    NW = info.num_cores * info.num_subcores
    fn = _make_sc_gather(B, d, NW, info.num_cores, C=32)
    out = fn(x.reshape(B), table)
    return out.reshape(batch, seq, d)


# speedup vs baseline: 2.3831x; 1.2002x over previous
import functools

import jax
import jax.numpy as jnp
from jax import lax
from jax.experimental import pallas as pl
from jax.experimental.pallas import tpu as pltpu
from jax.experimental.pallas import tpu_sc as plsc


def _make_sc_gather(B, D, NW, NC, C):
    b_per_w = B // NW
    nchunks = b_per_w // C
    assert nchunks >= 4 and nchunks % 2 == 0
    mesh = plsc.VectorSubcoreMesh(core_axis_name="c", subcore_axis_name="s")

    @functools.partial(
        pl.kernel,
        mesh=mesh,
        out_type=jax.ShapeDtypeStruct((B, D), jnp.float32),
        scratch_types=[
            pltpu.VMEM((b_per_w,), jnp.int32),
            pltpu.VMEM((C, D), jnp.float32),
            pltpu.VMEM((C, D), jnp.float32),
            pltpu.SemaphoreType.DMA,
            pltpu.SemaphoreType.DMA,
            pltpu.SemaphoreType.DMA,
            pltpu.SemaphoreType.DMA,
        ],
    )
    def k(idx_hbm, table_hbm, out_hbm, idx_v, buf0, buf1, g0, g1, s0, s1):
        wid = lax.axis_index("s") * NC + lax.axis_index("c")
        base = wid * b_per_w
        pltpu.sync_copy(idx_hbm.at[pl.ds(base, b_per_w)], idx_v)
        bufs = (buf0, buf1)
        gsems = (g0, g1)
        ssems = (s0, s1)

        def start_gather(c, b):
            pltpu.async_copy(
                table_hbm.at[idx_v.at[pl.ds(c * C, C)]], bufs[b], gsems[b]
            )

        def wait_gather(b):
            pltpu.make_async_copy(
                table_hbm.at[idx_v.at[pl.ds(0, C)]], bufs[b], gsems[b]
            ).wait()

        def start_scatter(c, b):
            pltpu.async_copy(bufs[b], out_hbm.at[pl.ds(base + c * C, C)], ssems[b])

        def wait_scatter(b):
            pltpu.make_async_copy(
                bufs[b], out_hbm.at[pl.ds(base, C)], ssems[b]
            ).wait()

        start_gather(0, 0)
        start_gather(1, 1)
        wait_gather(0)
        start_scatter(0, 0)

        def pair(g, carry):
            for par in range(2):
                c = 2 * g + 1 + par
                cur = (1 + par) % 2
                nxt = par % 2
                wait_scatter(nxt)
                start_gather(c + 1, nxt)
                wait_gather(cur)
                start_scatter(c, cur)
            return carry

        lax.fori_loop(0, (nchunks - 2) // 2, pair, 0)

        c = nchunks - 1
        wait_scatter((c + 1) % 2)
        wait_gather(c % 2)
        start_scatter(c, c % 2)
        wait_scatter(c % 2)

    return k


def kernel(x, table):
    batch, seq = x.shape
    max_len, d = table.shape
    B = batch * seq
    info = plsc.get_sparse_core_info()
    NW = info.num_cores * info.num_subcores
    fn = _make_sc_gather(B, d, NW, info.num_cores, C=32)
    out = fn(x.reshape(B), table)
    return out.reshape(batch, seq, d)
